# baseline (speedup 1.0000x reference)
import functools

import jax
import jax.numpy as jnp
from jax.experimental import pallas as pl
from jax.experimental.pallas import tpu as pltpu


def _fused_body(seg_ref, x_ref, c1_ref, c2_ref, Wenc_ref, benc_ref,
                Wt1_ref, bt1_ref, Wt2_ref, bt2_ref, temb_ref, Wc_ref, bc_ref,
                out_ref, q_scr, tp_scr, *, s, f, tn):
    b = pl.program_id(0)
    p = pl.program_id(1)
    d = x_ref.shape[-1]

    @pl.when(p == 0)
    def _pool_and_mlp():
        TE = temb_ref.shape[1]
        x = x_ref[0].reshape(s * f, d)
        f_lane = jax.lax.rem(
            jax.lax.broadcasted_iota(jnp.int32, (1, s * f), 1), f)
        l_lane = jnp.concatenate(
            [jnp.full((1, f), seg_ref[b, j], jnp.int32) for j in range(s)],
            axis=1)
        mask = (f_lane < l_lane).astype(jnp.float32)
        sums = jnp.dot(mask, x, preferred_element_type=jnp.float32)
        count = jnp.sum(l_lane.astype(jnp.float32)) * (1.0 / f)
        pooled = sums / count
        vf = jax.nn.relu(
            jnp.dot(pooled, Wenc_ref[...], preferred_element_type=jnp.float32)
            + benc_ref[...])
        cat = jnp.concatenate([c1_ref[0], c2_ref[0], vf], axis=1)
        h = jax.nn.relu(
            jnp.dot(cat, Wt1_ref[...], preferred_element_type=jnp.float32)
            + bt1_ref[...])
        logits = (jnp.dot(h, Wt2_ref[...], preferred_element_type=jnp.float32)
                  + bt2_ref[...])
        m = jnp.max(logits, axis=1, keepdims=True)
        e = jnp.exp(logits - m)
        tp_scr[...] = e / jnp.sum(e, axis=1, keepdims=True)
        E_T = jax.lax.dot_general(Wc_ref[0:TE, :], temb_ref[...],
                                  dimension_numbers=(((0,), (1,)), ((), ())),
                                  preferred_element_type=jnp.float32)
        catv = jnp.concatenate([vf, c1_ref[0], c2_ref[0]], axis=1)
        V = jax.lax.dot_general(Wc_ref[TE:, :], catv,
                                dimension_numbers=(((0,), (1,)), ((), ())),
                                preferred_element_type=jnp.float32)
        q_scr[...] = jax.nn.relu(E_T + V + bc_ref[...]).astype(jnp.bfloat16)

    @pl.when(p == 1)
    def _score():
        x = x_ref[0].reshape(s * f, d).astype(jnp.bfloat16)
        L = jnp.dot(x, q_scr[...], preferred_element_type=jnp.float32)
        sc = jax.nn.sigmoid(L) * tp_scr[...]
        sc = jax.nn.relu(sc - 0.01)
        tot = jnp.sum(sc, axis=1, keepdims=True) * (1.0 / tn)
        f_sub = jax.lax.rem(
            jax.lax.broadcasted_iota(jnp.int32, (s * f, 1), 0), f)
        l_sub = jnp.concatenate(
            [jnp.full((f, 1), seg_ref[b, j], jnp.int32) for j in range(s)],
            axis=0)
        tot = jnp.where(f_sub < l_sub, tot, 0.0)
        out_ref[...] = tot.reshape(1, s, f, 1)


def kernel(batch, seg_len, concept1, concept2, W_enc, b_enc, W_t1, b_t1,
           W_t2, b_t2, topic_emb, W_c, b_c):
    B, S, F, D = batch.shape
    TN, TE = topic_emb.shape
    SH = W_enc.shape[1]
    CD = concept1.shape[1]

    seg_len = seg_len.astype(jnp.int32)

    const = lambda *idx: (lambda b, p, seg: idx)

    overall = pl.pallas_call(
        functools.partial(_fused_body, s=S, f=F, tn=float(TN)),
        grid_spec=pltpu.PrefetchScalarGridSpec(
            num_scalar_prefetch=1,
            grid=(B, 2),
            in_specs=[
                pl.BlockSpec((1, S, F, D), lambda b, p, seg: (b, 0, 0, 0)),
                pl.BlockSpec((1, 1, CD), lambda b, p, seg: (b, 0, 0)),
                pl.BlockSpec((1, 1, CD), lambda b, p, seg: (b, 0, 0)),
                pl.BlockSpec((D, SH), const(0, 0)),
                pl.BlockSpec((1, SH), const(0, 0)),
                pl.BlockSpec(W_t1.shape, const(0, 0)),
                pl.BlockSpec((1, W_t1.shape[1]), const(0, 0)),
                pl.BlockSpec(W_t2.shape, const(0, 0)),
                pl.BlockSpec((1, TN), const(0, 0)),
                pl.BlockSpec((TN, TE), const(0, 0)),
                pl.BlockSpec(W_c.shape, const(0, 0)),
                pl.BlockSpec((D, 1), const(0, 0)),
            ],
            out_specs=pl.BlockSpec((1, S, F, 1), lambda b, p, seg: (b, 0, 0, 0)),
            scratch_shapes=[
                pltpu.VMEM((D, TN), jnp.bfloat16),
                pltpu.VMEM((1, TN), jnp.float32),
            ],
        ),
        out_shape=jax.ShapeDtypeStruct((B, S, F, 1), jnp.float32),
        compiler_params=pltpu.CompilerParams(
            dimension_semantics=("parallel", "arbitrary")),
    )(seg_len, batch, concept1.reshape(B, 1, CD), concept2.reshape(B, 1, CD),
      W_enc, b_enc.reshape(1, SH), W_t1, b_t1.reshape(1, -1),
      W_t2, b_t2.reshape(1, TN), topic_emb, W_c, b_c.reshape(D, 1))

    overall = overall.reshape(B, S, F)
    return (overall, overall)


# speedup vs baseline: 1.0006x; 1.0006x over previous
import functools

import jax
import jax.numpy as jnp
from jax.experimental import pallas as pl
from jax.experimental.pallas import tpu as pltpu

_K = 4


def _fused_body(seg_ref, *refs, s, f, tn, k):
    x_refs = refs[:k]
    (c1_ref, c2_ref, Wenc_ref, benc_ref, Wt1_ref, bt1_ref, Wt2_ref, bt2_ref,
     temb_ref, Wc_ref, bc_ref) = refs[k:k + 11]
    out_ref = refs[k + 11]
    q_scr, tp_scr = refs[k + 12:]
    b = pl.program_id(0)
    p = pl.program_id(1)
    sk = s // k
    d = x_refs[0].shape[-1]

    @pl.when(p == 0)
    def _pool_and_mlp():
        TE = temb_ref.shape[1]
        f_lane = jax.lax.rem(
            jax.lax.broadcasted_iota(jnp.int32, (1, sk * f), 1), f)
        sums = jnp.zeros((1, d), jnp.float32)
        for kk in range(k):
            x = x_refs[kk][0].reshape(sk * f, d)
            l_lane = jnp.concatenate(
                [jnp.full((1, f), seg_ref[b, kk * sk + j], jnp.int32)
                 for j in range(sk)], axis=1)
            mask = (f_lane < l_lane).astype(jnp.float32)
            sums = sums + jnp.dot(mask, x, preferred_element_type=jnp.float32)
        count = jnp.float32(0)
        for j in range(s):
            count = count + seg_ref[b, j].astype(jnp.float32)
        pooled = sums / count
        vf = jax.nn.relu(
            jnp.dot(pooled, Wenc_ref[...], preferred_element_type=jnp.float32)
            + benc_ref[...])
        cat = jnp.concatenate([c1_ref[0], c2_ref[0], vf], axis=1)
        h = jax.nn.relu(
            jnp.dot(cat, Wt1_ref[...], preferred_element_type=jnp.float32)
            + bt1_ref[...])
        logits = (jnp.dot(h, Wt2_ref[...], preferred_element_type=jnp.float32)
                  + bt2_ref[...])
        m = jnp.max(logits, axis=1, keepdims=True)
        e = jnp.exp(logits - m)
        tp_scr[...] = e / jnp.sum(e, axis=1, keepdims=True)
        E_T = jax.lax.dot_general(Wc_ref[0:TE, :], temb_ref[...],
                                  dimension_numbers=(((0,), (1,)), ((), ())),
                                  preferred_element_type=jnp.float32)
        catv = jnp.concatenate([vf, c1_ref[0], c2_ref[0]], axis=1)
        V = jax.lax.dot_general(Wc_ref[TE:, :], catv,
                                dimension_numbers=(((0,), (1,)), ((), ())),
                                preferred_element_type=jnp.float32)
        q_scr[...] = jax.nn.relu(E_T + V + bc_ref[...]).astype(jnp.bfloat16)

    @pl.when(p == 1)
    def _score():
        q = q_scr[...]
        tp = tp_scr[...]
        f_sub = jax.lax.rem(
            jax.lax.broadcasted_iota(jnp.int32, (sk * f, 1), 0), f)
        for kk in range(k):
            x = x_refs[kk][0].reshape(sk * f, d).astype(jnp.bfloat16)
            L = jnp.dot(x, q, preferred_element_type=jnp.float32)
            sc = jax.nn.sigmoid(L) * tp
            sc = jax.nn.relu(sc - 0.01)
            tot = jnp.sum(sc, axis=1, keepdims=True) * (1.0 / tn)
            l_sub = jnp.concatenate(
                [jnp.full((f, 1), seg_ref[b, kk * sk + j], jnp.int32)
                 for j in range(sk)], axis=0)
            tot = jnp.where(f_sub < l_sub, tot, 0.0)
            out_ref[0, kk * sk:(kk + 1) * sk] = tot.reshape(sk, f, 1)


def kernel(batch, seg_len, concept1, concept2, W_enc, b_enc, W_t1, b_t1,
           W_t2, b_t2, topic_emb, W_c, b_c):
    B, S, F, D = batch.shape
    TN, TE = topic_emb.shape
    SH = W_enc.shape[1]
    CD = concept1.shape[1]
    K = _K if S % _K == 0 else 1
    SK = S // K

    seg_len = seg_len.astype(jnp.int32)

    const = lambda *idx: (lambda b, p, seg: idx)
    x_specs = [
        pl.BlockSpec((1, SK, F, D),
                     lambda b, p, seg, kk=kk: (b, kk, 0, 0))
        for kk in range(K)
    ]

    overall = pl.pallas_call(
        functools.partial(_fused_body, s=S, f=F, tn=float(TN), k=K),
        grid_spec=pltpu.PrefetchScalarGridSpec(
            num_scalar_prefetch=1,
            grid=(B, 2),
            in_specs=x_specs + [
                pl.BlockSpec((1, 1, CD), lambda b, p, seg: (b, 0, 0)),
                pl.BlockSpec((1, 1, CD), lambda b, p, seg: (b, 0, 0)),
                pl.BlockSpec((D, SH), const(0, 0)),
                pl.BlockSpec((1, SH), const(0, 0)),
                pl.BlockSpec(W_t1.shape, const(0, 0)),
                pl.BlockSpec((1, W_t1.shape[1]), const(0, 0)),
                pl.BlockSpec(W_t2.shape, const(0, 0)),
                pl.BlockSpec((1, TN), const(0, 0)),
                pl.BlockSpec((TN, TE), const(0, 0)),
                pl.BlockSpec(W_c.shape, const(0, 0)),
                pl.BlockSpec((D, 1), const(0, 0)),
            ],
            out_specs=pl.BlockSpec((1, S, F, 1), lambda b, p, seg: (b, 0, 0, 0)),
            scratch_shapes=[
                pltpu.VMEM((D, TN), jnp.bfloat16),
                pltpu.VMEM((1, TN), jnp.float32),
            ],
        ),
        out_shape=jax.ShapeDtypeStruct((B, S, F, 1), jnp.float32),
        compiler_params=pltpu.CompilerParams(
            dimension_semantics=("parallel", "arbitrary")),
    )(seg_len, *([batch] * K), concept1.reshape(B, 1, CD),
      concept2.reshape(B, 1, CD), W_enc, b_enc.reshape(1, SH), W_t1,
      b_t1.reshape(1, -1), W_t2, b_t2.reshape(1, TN), topic_emb, W_c,
      b_c.reshape(D, 1))

    overall = overall.reshape(B, S, F)
    return (overall, overall)


# speedup vs baseline: 1.1609x; 1.1602x over previous
import functools

import jax
import jax.numpy as jnp
from jax.experimental import pallas as pl
from jax.experimental.pallas import tpu as pltpu

_K = 4


def _fused_body(seg_ref, *refs, s, f, tn, k):
    x_refs = refs[:k]
    (c1_ref, c2_ref, Wenc_ref, benc_ref, Wt1_ref, bt1_ref, Wt2_ref, bt2_ref,
     temb_ref, Wc_ref, bc_ref) = refs[k:k + 11]
    out_ref = refs[k + 11]
    q_scr, tp_scr = refs[k + 12:]
    b = pl.program_id(0)
    p = pl.program_id(1)
    sk = s // k
    d = x_refs[0].shape[-1]

    @pl.when(p == 0)
    def _pool_and_mlp():
        TE = temb_ref.shape[1]
        f_lane = jax.lax.rem(
            jax.lax.broadcasted_iota(jnp.int32, (1, sk * f), 1), f)
        sums = jnp.zeros((1, d), jnp.float32)
        for kk in range(k):
            x = x_refs[kk][0].reshape(sk * f, d)
            l_lane = jnp.concatenate(
                [jnp.full((1, f), seg_ref[b, kk * sk + j], jnp.int32)
                 for j in range(sk)], axis=1)
            mask = (f_lane < l_lane).astype(jnp.float32)
            sums = sums + jnp.dot(mask, x, preferred_element_type=jnp.float32)
        count = jnp.float32(0)
        for j in range(s):
            count = count + seg_ref[b, j].astype(jnp.float32)
        pooled = sums / count
        vf = jax.nn.relu(
            jnp.dot(pooled, Wenc_ref[...], preferred_element_type=jnp.float32)
            + benc_ref[...])
        cat = jnp.concatenate([c1_ref[0], c2_ref[0], vf], axis=1)
        h = jax.nn.relu(
            jnp.dot(cat, Wt1_ref[...], preferred_element_type=jnp.float32)
            + bt1_ref[...])
        logits = (jnp.dot(h, Wt2_ref[...], preferred_element_type=jnp.float32)
                  + bt2_ref[...])
        m = jnp.max(logits, axis=1, keepdims=True)
        e = jnp.exp(logits - m)
        tp_scr[...] = e / jnp.sum(e, axis=1, keepdims=True)
        E_T = jax.lax.dot_general(Wc_ref[0:TE, :], temb_ref[...],
                                  dimension_numbers=(((0,), (1,)), ((), ())),
                                  preferred_element_type=jnp.float32)
        catv = jnp.concatenate([vf, c1_ref[0], c2_ref[0]], axis=1)
        V = jax.lax.dot_general(Wc_ref[TE:, :], catv,
                                dimension_numbers=(((0,), (1,)), ((), ())),
                                preferred_element_type=jnp.float32)
        q_scr[...] = jax.nn.relu(E_T + V + bc_ref[...]).astype(jnp.bfloat16)

    @pl.when(p == 1)
    def _score():
        q = q_scr[...]
        tp = tp_scr[...]
        f_sub = jax.lax.rem(
            jax.lax.broadcasted_iota(jnp.int32, (sk * f, 1), 0), f)
        for kk in range(k):
            x = x_refs[kk][0].reshape(sk * f, d).astype(jnp.bfloat16)
            L = jnp.dot(x, q, preferred_element_type=jnp.float32)
            sc = jax.nn.sigmoid(L) * tp
            sc = jax.nn.relu(sc - 0.01)
            tot = jnp.sum(sc, axis=1, keepdims=True) * (1.0 / tn)
            l_sub = jnp.concatenate(
                [jnp.full((f, 1), seg_ref[b, kk * sk + j], jnp.int32)
                 for j in range(sk)], axis=0)
            tot = jnp.where(f_sub < l_sub, tot, 0.0)
            out_ref[0, kk * sk:(kk + 1) * sk] = tot.reshape(sk, f)


def kernel(batch, seg_len, concept1, concept2, W_enc, b_enc, W_t1, b_t1,
           W_t2, b_t2, topic_emb, W_c, b_c):
    B, S, F, D = batch.shape
    TN, TE = topic_emb.shape
    SH = W_enc.shape[1]
    CD = concept1.shape[1]
    K = _K if S % _K == 0 else 1
    SK = S // K

    seg_len = seg_len.astype(jnp.int32)

    const = lambda *idx: (lambda b, p, seg: idx)
    x_specs = [
        pl.BlockSpec((1, SK, F, D),
                     lambda b, p, seg, kk=kk: (b, kk, 0, 0))
        for kk in range(K)
    ]

    overall = pl.pallas_call(
        functools.partial(_fused_body, s=S, f=F, tn=float(TN), k=K),
        grid_spec=pltpu.PrefetchScalarGridSpec(
            num_scalar_prefetch=1,
            grid=(B, 2),
            in_specs=x_specs + [
                pl.BlockSpec((1, 1, CD), lambda b, p, seg: (b, 0, 0)),
                pl.BlockSpec((1, 1, CD), lambda b, p, seg: (b, 0, 0)),
                pl.BlockSpec((D, SH), const(0, 0)),
                pl.BlockSpec((1, SH), const(0, 0)),
                pl.BlockSpec(W_t1.shape, const(0, 0)),
                pl.BlockSpec((1, W_t1.shape[1]), const(0, 0)),
                pl.BlockSpec(W_t2.shape, const(0, 0)),
                pl.BlockSpec((1, TN), const(0, 0)),
                pl.BlockSpec((TN, TE), const(0, 0)),
                pl.BlockSpec(W_c.shape, const(0, 0)),
                pl.BlockSpec((D, 1), const(0, 0)),
            ],
            out_specs=pl.BlockSpec((1, S, F), lambda b, p, seg: (b, 0, 0)),
            scratch_shapes=[
                pltpu.VMEM((D, TN), jnp.bfloat16),
                pltpu.VMEM((1, TN), jnp.float32),
            ],
        ),
        out_shape=jax.ShapeDtypeStruct((B, S, F), jnp.float32),
        compiler_params=pltpu.CompilerParams(
            dimension_semantics=("parallel", "arbitrary")),
    )(seg_len, *([batch] * K), concept1.reshape(B, 1, CD),
      concept2.reshape(B, 1, CD), W_enc, b_enc.reshape(1, SH), W_t1,
      b_t1.reshape(1, -1), W_t2, b_t2.reshape(1, TN), topic_emb, W_c,
      b_c.reshape(D, 1))

    return (overall, overall)
